# baseline (speedup 1.0000x reference)
import functools

import jax
import jax.numpy as jnp
from jax import lax
from jax.experimental import pallas as pl
from jax.experimental.pallas import tpu as pltpu
from jax.experimental.pallas import tpu_sc as plsc

NN = 10000
EE = 320000
DD = 128
OUTD = 64
FFD = 2048

NC = 2
NS = 16
NW = NC * NS
CHK = 128
GCH = 79
EWP = GCH * CHK
EP = NW * EWP
NN2 = 10048
NBUF = 3
NBUF_DEG = 2
RPS = 624
TAIL = NN - NS * RPS

BN = 1000
GN = NN // BN
FCH = 512


def _agg_body(want_deg, h_hbm, z_hbm, src_hbm, dst_hbm, *rest):
    if want_deg:
        (out_hbm, deg_hbm, idx_s, idx_d, rows, accum, degp, gsem, isem) = rest
        nbuf = NBUF_DEG
    else:
        (out_hbm, idx_s, idx_d, rows, accum, gsem, isem) = rest
        nbuf = NBUF
    idxr = nbuf + 2
    c = lax.axis_index("c")
    s = lax.axis_index("s")
    wid = c * NS + s
    @pl.when(c == 0)
    def _():
        pltpu.sync_copy(h_hbm.at[pl.ds(s * RPS, RPS)],
                        accum.at[pl.ds(s * RPS, RPS)])
        @pl.when(s == NS - 1)
        def _():
            pltpu.sync_copy(h_hbm.at[pl.ds(NS * RPS, TAIL)],
                            accum.at[pl.ds(NS * RPS, TAIL)])
    @pl.when(c != 0)
    def _():
        pltpu.sync_copy(z_hbm.at[pl.ds(s * RPS, RPS)],
                        accum.at[pl.ds(s * RPS, RPS)])
        @pl.when(s == NS - 1)
        def _():
            pltpu.sync_copy(z_hbm.at[pl.ds(NS * RPS, TAIL)],
                            accum.at[pl.ds(NS * RPS, TAIL)])
    plsc.subcore_barrier()

    if want_deg:
        zeros16 = jnp.zeros((16,), jnp.float32)
        ones16 = jnp.ones((16,), jnp.float32)

        def zbody(i, _):
            degp[pl.ds(i * 16, 16)] = zeros16
            return 0

        lax.fori_loop(0, NN2 // 16, zbody, 0)

    def _idx_load(q):
        slot = lax.rem(q, idxr)
        pltpu.async_copy(src_hbm.at[wid, q], idx_s.at[slot], isem)
        pltpu.async_copy(dst_hbm.at[wid, q], idx_d.at[slot], isem)

    def _idx_wait(q):
        slot = lax.rem(q, idxr)
        pltpu.make_async_copy(src_hbm.at[wid, q], idx_s.at[slot], isem).wait()
        pltpu.make_async_copy(dst_hbm.at[wid, q], idx_d.at[slot], isem).wait()

    def _gather(q):
        pltpu.async_copy(h_hbm.at[idx_s.at[lax.rem(q, idxr)]],
                         rows.at[lax.rem(q, nbuf)], gsem)

    for r in range(idxr - 1):
        _idx_load(r)
    for j in range(nbuf - 1):
        _idx_wait(j)
        _gather(j)

    def body(g, _):
        q = g + nbuf - 1
        @pl.when(q < GCH)
        def _():
            @pl.when(g + idxr - 1 < GCH)
            def _():
                _idx_load(g + idxr - 1)
            _idx_wait(q)
            _gather(q)
        pltpu.make_async_copy(h_hbm.at[idx_s.at[lax.rem(g, idxr)]],
                              rows.at[lax.rem(g, nbuf)], gsem).wait()
        dslot = lax.rem(g, idxr)
        if want_deg:
            for k in range(CHK // 16):
                v = idx_d[dslot, pl.ds(k * 16, 16)]
                plsc.addupdate_scatter(degp, [v], ones16)
        pltpu.sync_copy(rows.at[lax.rem(g, nbuf)], accum.at[idx_d.at[dslot]],
                        add=True)
        return 0

    lax.fori_loop(0, GCH, body, 0)

    if want_deg:
        pltpu.sync_copy(degp.at[pl.ds(0, NN)], deg_hbm.at[pl.ds(wid * NN, NN)])

    plsc.subcore_barrier()
    pltpu.sync_copy(accum.at[pl.ds(s * RPS, RPS)],
                    out_hbm.at[c, pl.ds(s * RPS, RPS)])
    @pl.when(s == NS - 1)
    def _():
        pltpu.sync_copy(accum.at[pl.ds(NS * RPS, TAIL)],
                        out_hbm.at[c, pl.ds(NS * RPS, TAIL)])


@functools.lru_cache(maxsize=2)
def _build_agg_sc(want_deg):
    mesh = plsc.VectorSubcoreMesh(
        core_axis_name="c", subcore_axis_name="s",
        num_cores=NC, num_subcores=NS)
    nbuf = NBUF_DEG if want_deg else NBUF
    idxr = nbuf + 2
    out_type = [jax.ShapeDtypeStruct((NC, NN, DD), jnp.float32)]
    scratch = [
        pltpu.VMEM((idxr, CHK), jnp.int32),
        pltpu.VMEM((idxr, CHK), jnp.int32),
        pltpu.VMEM((nbuf, CHK, DD), jnp.float32),
        pltpu.VMEM_SHARED((NN2, DD), jnp.float32),
    ]
    if want_deg:
        out_type.append(jax.ShapeDtypeStruct((NW * NN,), jnp.float32))
        scratch.append(pltpu.VMEM((NN2,), jnp.float32))
    scratch.append(pltpu.SemaphoreType.DMA)
    scratch.append(pltpu.SemaphoreType.DMA)
    return pl.kernel(
        functools.partial(_agg_body, want_deg),
        out_type=out_type,
        mesh=mesh,
        scratch_types=scratch,
        compiler_params=pltpu.CompilerParams(
            use_tc_tiling_on_sc=False, needs_layout_passes=False),
    )


def _agg_call(h, zeros_nd, src_r, dst_r, want_deg):
    return _build_agg_sc(want_deg)(h, zeros_nd, src_r, dst_r)


def _dotT(x, w):
    return lax.dot_general(x, w, (((1,), (1,)), ((), ())),
                           preferred_element_type=jnp.float32)


def _dot(x, w):
    return lax.dot_general(x, w, (((1,), (0,)), ((), ())),
                           preferred_element_type=jnp.float32)


def _dense_in_body(feat_ref, w1_ref, b1_ref, w2_ref, out_ref):
    t = _dotT(feat_ref[...], w1_ref[...]) + b1_ref[...]
    out_ref[...] = _dot(t, w2_ref[...])


_dense_in = pl.pallas_call(
    _dense_in_body,
    grid=(GN,),
    in_specs=[
        pl.BlockSpec((BN, DD), lambda i: (i, 0)),
        pl.BlockSpec((DD, DD), lambda i: (0, 0)),
        pl.BlockSpec((1, DD), lambda i: (0, 0)),
        pl.BlockSpec((DD, DD), lambda i: (0, 0)),
    ],
    out_specs=pl.BlockSpec((BN, DD), lambda i: (i, 0)),
    out_shape=jax.ShapeDtypeStruct((NN, DD), jnp.float32),
)


def _layer_body(parts_ref, deg_ref, gw_ref, gb_ref, a_ref, aw_ref, ab_ref,
                h_ref, tsum_ref):
    p = parts_ref[0] + parts_ref[1]
    deg = jnp.sum(deg_ref[0], axis=1, keepdims=True) + 1.0
    hc = _dot(p * (1.0 / deg), gw_ref[...]) + gb_ref[...]
    hc = jnp.where(hc >= 0, hc, a_ref[...] * hc)
    t = jnp.tanh(_dotT(hc, aw_ref[...]) + ab_ref[...])
    tsum_ref[...] = jnp.sum(t, axis=0, keepdims=True).reshape(1, 1, DD)
    h_ref[...] = hc


_layer = pl.pallas_call(
    _layer_body,
    grid=(GN,),
    in_specs=[
        pl.BlockSpec((NC, BN, DD), lambda i: (0, i, 0)),
        pl.BlockSpec((1, BN, NW), lambda i: (i, 0, 0)),
        pl.BlockSpec((DD, DD), lambda i: (0, 0)),
        pl.BlockSpec((1, DD), lambda i: (0, 0)),
        pl.BlockSpec((1, DD), lambda i: (0, 0)),
        pl.BlockSpec((DD, DD), lambda i: (0, 0)),
        pl.BlockSpec((1, DD), lambda i: (0, 0)),
    ],
    out_specs=[
        pl.BlockSpec((BN, DD), lambda i: (i, 0)),
        pl.BlockSpec((1, 1, DD), lambda i: (i, 0, 0)),
    ],
    out_shape=[
        jax.ShapeDtypeStruct((NN, DD), jnp.float32),
        jax.ShapeDtypeStruct((GN, 1, DD), jnp.float32),
    ],
)


def _ln(x, w, b):
    mu = jnp.mean(x, axis=-1, keepdims=True)
    var = jnp.mean((x - mu) ** 2, axis=-1, keepdims=True)
    return (x - mu) / jnp.sqrt(var + 1e-5) * w + b


def _tail_body(h1_ref, h2_ref, ts1_ref, ts2_ref, av_ref,
               wv_sa_ref, bv_sa_ref, ow_sa_ref, ob_sa_ref,
               wv_ca_ref, bv_ca_ref, ow_ca_ref, ob_ca_ref,
               ln1w_ref, ln1b_ref, ln2w_ref, ln2b_ref, ln3w_ref, ln3b_ref,
               ff1_ref, ff1b_ref, ff2_ref, ff2b_ref,
               pw_ref, pb_ref, out_ref):
    av = av_ref[...]
    s1 = jnp.sum(jnp.sum(ts1_ref[...], axis=0) * av) / NN
    s2 = jnp.sum(jnp.sum(ts2_ref[...], axis=0) * av) / NN
    mx = jnp.maximum(s1, s2)
    e1 = jnp.exp(s1 - mx)
    e2 = jnp.exp(s2 - mx)
    b0 = e1 / (e1 + e2)
    b1 = e2 / (e1 + e2)
    z = b0 * h1_ref[...] + b1 * h2_ref[...]
    sa = _dotT(_dotT(z, wv_sa_ref[...]) + bv_sa_ref[...],
               ow_sa_ref[...]) + ob_sa_ref[...]
    x1 = _ln(z + sa, ln1w_ref[...], ln1b_ref[...])
    ca = _dotT(_dotT(z, wv_ca_ref[...]) + bv_ca_ref[...],
               ow_ca_ref[...]) + ob_ca_ref[...]
    x2 = _ln(x1 + ca, ln2w_ref[...], ln2b_ref[...])
    ff = jnp.zeros((BN, DD), jnp.float32) + ff2b_ref[...]
    x2b = x2.astype(jnp.bfloat16)
    for k in range(FFD // FCH):
        w1c = ff1_ref[k * FCH:(k + 1) * FCH, :]
        b1c = ff1b_ref[:, k * FCH:(k + 1) * FCH]
        hck = jnp.maximum(_dotT(x2b, w1c) + b1c, 0.0)
        ff = ff + _dotT(hck.astype(jnp.bfloat16),
                        ff2_ref[:, k * FCH:(k + 1) * FCH])
    x3 = _ln(x2 + ff, ln3w_ref[...], ln3b_ref[...])
    out_ref[...] = _dotT(x3, pw_ref[...]) + pb_ref[...]


_tail = pl.pallas_call(
    _tail_body,
    grid=(GN,),
    in_specs=[
        pl.BlockSpec((BN, DD), lambda i: (i, 0)),
        pl.BlockSpec((BN, DD), lambda i: (i, 0)),
        pl.BlockSpec((GN, 1, DD), lambda i: (0, 0, 0)),
        pl.BlockSpec((GN, 1, DD), lambda i: (0, 0, 0)),
        pl.BlockSpec((1, DD), lambda i: (0, 0)),
        pl.BlockSpec((DD, DD), lambda i: (0, 0)),
        pl.BlockSpec((1, DD), lambda i: (0, 0)),
        pl.BlockSpec((DD, DD), lambda i: (0, 0)),
        pl.BlockSpec((1, DD), lambda i: (0, 0)),
        pl.BlockSpec((DD, DD), lambda i: (0, 0)),
        pl.BlockSpec((1, DD), lambda i: (0, 0)),
        pl.BlockSpec((DD, DD), lambda i: (0, 0)),
        pl.BlockSpec((1, DD), lambda i: (0, 0)),
        pl.BlockSpec((1, DD), lambda i: (0, 0)),
        pl.BlockSpec((1, DD), lambda i: (0, 0)),
        pl.BlockSpec((1, DD), lambda i: (0, 0)),
        pl.BlockSpec((1, DD), lambda i: (0, 0)),
        pl.BlockSpec((1, DD), lambda i: (0, 0)),
        pl.BlockSpec((1, DD), lambda i: (0, 0)),
        pl.BlockSpec((FFD, DD), lambda i: (0, 0)),
        pl.BlockSpec((1, FFD), lambda i: (0, 0)),
        pl.BlockSpec((DD, FFD), lambda i: (0, 0)),
        pl.BlockSpec((1, DD), lambda i: (0, 0)),
        pl.BlockSpec((OUTD, DD), lambda i: (0, 0)),
        pl.BlockSpec((1, OUTD), lambda i: (0, 0)),
    ],
    out_specs=pl.BlockSpec((BN, OUTD), lambda i: (i, 0)),
    out_shape=jax.ShapeDtypeStruct((NN, OUTD), jnp.float32),
)


def kernel(feat, params, edge_index):
    p = params
    row = lambda v: v.reshape(1, -1)
    pad = EP - EE
    pad_src = (jnp.arange(pad, dtype=jnp.int32) * 613) % NN
    pad_dst = NN + jnp.arange(pad, dtype=jnp.int32) % (NN2 - NN)
    src_r = jnp.concatenate([edge_index[0], pad_src]).reshape(NW, GCH, CHK)
    dst_r = jnp.concatenate([edge_index[1], pad_dst]).reshape(NW, GCH, CHK)
    zeros_nd = jnp.zeros((NN, DD), jnp.float32)

    h0 = _dense_in(feat, p["fc_in_w"], row(p["fc_in_b"]), p["w_cites"])

    parts0, deg_flat = _agg_call(h0, zeros_nd, src_r, dst_r, True)
    deg_r = deg_flat.reshape(NW, NN).T.reshape(GN, BN, NW)
    h1, ts1 = _layer(parts0, deg_r, p["gc_w"][0], row(p["gc_b"][0]),
                     jnp.full((1, DD), p["prelu_a"][0], jnp.float32),
                     p["attn_fc_w"], row(p["attn_fc_b"]))

    (parts1,) = _agg_call(h1, zeros_nd, src_r, dst_r, False)
    h2, ts2 = _layer(parts1, deg_r, p["gc_w"][1], row(p["gc_b"][1]),
                     jnp.full((1, DD), p["prelu_a"][1], jnp.float32),
                     p["attn_fc_w"], row(p["attn_fc_b"]))

    out = _tail(h1, h2, ts1, ts2, p["attn_vec"],
                p["sa_in_w"][2 * DD:], row(p["sa_in_b"][2 * DD:]),
                p["sa_out_w"], row(p["sa_out_b"]),
                p["ca_in_w"][2 * DD:], row(p["ca_in_b"][2 * DD:]),
                p["ca_out_w"], row(p["ca_out_b"]),
                row(p["ln1_w"]), row(p["ln1_b"]),
                row(p["ln2_w"]), row(p["ln2_b"]),
                row(p["ln3_w"]), row(p["ln3_b"]),
                p["ff1_w"].astype(jnp.bfloat16), row(p["ff1_b"]),
                p["ff2_w"].astype(jnp.bfloat16), row(p["ff2_b"]),
                p["pred_w"], row(p["pred_b"]))
    return out


# speedup vs baseline: 1.0117x; 1.0117x over previous
import functools

import jax
import jax.numpy as jnp
from jax import lax
from jax.experimental import pallas as pl
from jax.experimental.pallas import tpu as pltpu
from jax.experimental.pallas import tpu_sc as plsc

NN = 10000
EE = 320000
DD = 128
OUTD = 64
FFD = 2048

NC = 2
NS = 16
NW = NC * NS
EW = EE // NW
CHK = 80
GCH = EW // CHK
NBUF = 4
NBUF_DEG = 3
RPS = 624
TAIL = NN - NS * RPS

BN = 1000
GN = NN // BN
FCH = 512


def _agg_body(want_deg, h_hbm, z_hbm, src_hbm, dst_hbm, *rest):
    if want_deg:
        (out_hbm, deg_hbm, idx_s, idx_d, rows, accum, degp,
         gsem, isem, ssem) = rest
        nbuf = NBUF_DEG
    else:
        (out_hbm, idx_s, idx_d, rows, accum, gsem, isem, ssem) = rest
        nbuf = NBUF
    idxr = nbuf + 2
    c = lax.axis_index("c")
    s = lax.axis_index("s")
    wid = c * NS + s
    @pl.when(c == 0)
    def _():
        pltpu.sync_copy(h_hbm.at[pl.ds(s * RPS, RPS)],
                        accum.at[pl.ds(s * RPS, RPS)])
        @pl.when(s == NS - 1)
        def _():
            pltpu.sync_copy(h_hbm.at[pl.ds(NS * RPS, TAIL)],
                            accum.at[pl.ds(NS * RPS, TAIL)])
    @pl.when(c != 0)
    def _():
        pltpu.sync_copy(z_hbm.at[pl.ds(s * RPS, RPS)],
                        accum.at[pl.ds(s * RPS, RPS)])
        @pl.when(s == NS - 1)
        def _():
            pltpu.sync_copy(z_hbm.at[pl.ds(NS * RPS, TAIL)],
                            accum.at[pl.ds(NS * RPS, TAIL)])
    plsc.subcore_barrier()

    if want_deg:
        zeros16 = jnp.zeros((16,), jnp.float32)
        ones16 = jnp.ones((16,), jnp.float32)

        def zbody(i, _):
            degp[pl.ds(i * 16, 16)] = zeros16
            return 0

        lax.fori_loop(0, NN // 16, zbody, 0)

    def _idx_load(q):
        slot = lax.rem(q, idxr)
        pltpu.async_copy(src_hbm.at[wid, q], idx_s.at[slot], isem)
        pltpu.async_copy(dst_hbm.at[wid, q], idx_d.at[slot], isem)

    def _idx_wait(q):
        slot = lax.rem(q, idxr)
        pltpu.make_async_copy(src_hbm.at[wid, q], idx_s.at[slot], isem).wait()
        pltpu.make_async_copy(dst_hbm.at[wid, q], idx_d.at[slot], isem).wait()

    def _gather(q):
        pltpu.async_copy(h_hbm.at[idx_s.at[lax.rem(q, idxr)]],
                         rows.at[lax.rem(q, nbuf)], gsem)

    for r in range(idxr - 1):
        _idx_load(r)
    for j in range(nbuf - 1):
        _idx_wait(j)
        _gather(j)

    def body(g, _):
        q = g + nbuf - 1
        pltpu.make_async_copy(h_hbm.at[idx_s.at[lax.rem(g, idxr)]],
                              rows.at[lax.rem(g, nbuf)], gsem).wait()
        dslot = lax.rem(g, idxr)
        pltpu.async_copy(rows.at[lax.rem(g, nbuf)], accum.at[idx_d.at[dslot]],
                         ssem, add=True)
        if want_deg:
            for k in range(CHK // 16):
                v = idx_d[dslot, pl.ds(k * 16, 16)]
                plsc.addupdate_scatter(degp, [v], ones16)
        @pl.when(q < GCH)
        def _():
            @pl.when(g + idxr - 1 < GCH)
            def _():
                _idx_load(g + idxr - 1)
            @pl.when(g >= 1)
            def _():
                pltpu.make_async_copy(
                    rows.at[0], accum.at[idx_d.at[0]], ssem).wait()
            _idx_wait(q)
            _gather(q)
        return 0

    lax.fori_loop(0, GCH, body, 0)
    for _ in range(nbuf):
        pltpu.make_async_copy(rows.at[0], accum.at[idx_d.at[0]], ssem).wait()

    if want_deg:
        pltpu.sync_copy(degp, deg_hbm.at[pl.ds(wid * NN, NN)])

    plsc.subcore_barrier()
    pltpu.sync_copy(accum.at[pl.ds(s * RPS, RPS)],
                    out_hbm.at[c, pl.ds(s * RPS, RPS)])
    @pl.when(s == NS - 1)
    def _():
        pltpu.sync_copy(accum.at[pl.ds(NS * RPS, TAIL)],
                        out_hbm.at[c, pl.ds(NS * RPS, TAIL)])


@functools.lru_cache(maxsize=2)
def _build_agg_sc(want_deg):
    mesh = plsc.VectorSubcoreMesh(
        core_axis_name="c", subcore_axis_name="s",
        num_cores=NC, num_subcores=NS)
    nbuf = NBUF_DEG if want_deg else NBUF
    idxr = nbuf + 2
    out_type = [jax.ShapeDtypeStruct((NC, NN, DD), jnp.float32)]
    scratch = [
        pltpu.VMEM((idxr, CHK), jnp.int32),
        pltpu.VMEM((idxr, CHK), jnp.int32),
        pltpu.VMEM((nbuf, CHK, DD), jnp.float32),
        pltpu.VMEM_SHARED((NN, DD), jnp.float32),
    ]
    if want_deg:
        out_type.append(jax.ShapeDtypeStruct((NW * NN,), jnp.float32))
        scratch.append(pltpu.VMEM((NN,), jnp.float32))
    scratch.append(pltpu.SemaphoreType.DMA)
    scratch.append(pltpu.SemaphoreType.DMA)
    scratch.append(pltpu.SemaphoreType.DMA)
    return pl.kernel(
        functools.partial(_agg_body, want_deg),
        out_type=out_type,
        mesh=mesh,
        scratch_types=scratch,
        compiler_params=pltpu.CompilerParams(
            use_tc_tiling_on_sc=False, needs_layout_passes=False),
    )


def _agg_call(h, zeros_nd, src_r, dst_r, want_deg):
    return _build_agg_sc(want_deg)(h, zeros_nd, src_r, dst_r)


def _dotT(x, w):
    return lax.dot_general(x, w, (((1,), (1,)), ((), ())),
                           preferred_element_type=jnp.float32)


def _dot(x, w):
    return lax.dot_general(x, w, (((1,), (0,)), ((), ())),
                           preferred_element_type=jnp.float32)


def _dense_in_body(feat_ref, w1_ref, b1_ref, w2_ref, out_ref):
    t = _dotT(feat_ref[...], w1_ref[...]) + b1_ref[...]
    out_ref[...] = _dot(t, w2_ref[...])


_dense_in = pl.pallas_call(
    _dense_in_body,
    grid=(GN,),
    in_specs=[
        pl.BlockSpec((BN, DD), lambda i: (i, 0)),
        pl.BlockSpec((DD, DD), lambda i: (0, 0)),
        pl.BlockSpec((1, DD), lambda i: (0, 0)),
        pl.BlockSpec((DD, DD), lambda i: (0, 0)),
    ],
    out_specs=pl.BlockSpec((BN, DD), lambda i: (i, 0)),
    out_shape=jax.ShapeDtypeStruct((NN, DD), jnp.float32),
)


def _layer_body(parts_ref, deg_ref, gw_ref, gb_ref, a_ref, aw_ref, ab_ref,
                h_ref, tsum_ref):
    p = parts_ref[0] + parts_ref[1]
    deg = jnp.sum(deg_ref[0], axis=1, keepdims=True) + 1.0
    hc = _dot(p * (1.0 / deg), gw_ref[...]) + gb_ref[...]
    hc = jnp.where(hc >= 0, hc, a_ref[...] * hc)
    t = jnp.tanh(_dotT(hc, aw_ref[...]) + ab_ref[...])
    tsum_ref[...] = jnp.sum(t, axis=0, keepdims=True).reshape(1, 1, DD)
    h_ref[...] = hc


_layer = pl.pallas_call(
    _layer_body,
    grid=(GN,),
    in_specs=[
        pl.BlockSpec((NC, BN, DD), lambda i: (0, i, 0)),
        pl.BlockSpec((1, BN, NW), lambda i: (i, 0, 0)),
        pl.BlockSpec((DD, DD), lambda i: (0, 0)),
        pl.BlockSpec((1, DD), lambda i: (0, 0)),
        pl.BlockSpec((1, DD), lambda i: (0, 0)),
        pl.BlockSpec((DD, DD), lambda i: (0, 0)),
        pl.BlockSpec((1, DD), lambda i: (0, 0)),
    ],
    out_specs=[
        pl.BlockSpec((BN, DD), lambda i: (i, 0)),
        pl.BlockSpec((1, 1, DD), lambda i: (i, 0, 0)),
    ],
    out_shape=[
        jax.ShapeDtypeStruct((NN, DD), jnp.float32),
        jax.ShapeDtypeStruct((GN, 1, DD), jnp.float32),
    ],
)


def _ln(x, w, b):
    mu = jnp.mean(x, axis=-1, keepdims=True)
    var = jnp.mean((x - mu) ** 2, axis=-1, keepdims=True)
    return (x - mu) / jnp.sqrt(var + 1e-5) * w + b


def _tail_body(h1_ref, h2_ref, ts1_ref, ts2_ref, av_ref,
               wv_sa_ref, bv_sa_ref, ow_sa_ref, ob_sa_ref,
               wv_ca_ref, bv_ca_ref, ow_ca_ref, ob_ca_ref,
               ln1w_ref, ln1b_ref, ln2w_ref, ln2b_ref, ln3w_ref, ln3b_ref,
               ff1_ref, ff1b_ref, ff2_ref, ff2b_ref,
               pw_ref, pb_ref, out_ref):
    av = av_ref[...]
    s1 = jnp.sum(jnp.sum(ts1_ref[...], axis=0) * av) / NN
    s2 = jnp.sum(jnp.sum(ts2_ref[...], axis=0) * av) / NN
    mx = jnp.maximum(s1, s2)
    e1 = jnp.exp(s1 - mx)
    e2 = jnp.exp(s2 - mx)
    b0 = e1 / (e1 + e2)
    b1 = e2 / (e1 + e2)
    z = b0 * h1_ref[...] + b1 * h2_ref[...]
    sa = _dotT(_dotT(z, wv_sa_ref[...]) + bv_sa_ref[...],
               ow_sa_ref[...]) + ob_sa_ref[...]
    x1 = _ln(z + sa, ln1w_ref[...], ln1b_ref[...])
    ca = _dotT(_dotT(z, wv_ca_ref[...]) + bv_ca_ref[...],
               ow_ca_ref[...]) + ob_ca_ref[...]
    x2 = _ln(x1 + ca, ln2w_ref[...], ln2b_ref[...])
    ff = jnp.zeros((BN, DD), jnp.float32) + ff2b_ref[...]
    x2b = x2.astype(jnp.bfloat16)
    for k in range(FFD // FCH):
        w1c = ff1_ref[k * FCH:(k + 1) * FCH, :]
        b1c = ff1b_ref[:, k * FCH:(k + 1) * FCH]
        hck = jnp.maximum(_dotT(x2b, w1c) + b1c, 0.0)
        ff = ff + _dotT(hck.astype(jnp.bfloat16),
                        ff2_ref[:, k * FCH:(k + 1) * FCH])
    x3 = _ln(x2 + ff, ln3w_ref[...], ln3b_ref[...])
    out_ref[...] = _dotT(x3, pw_ref[...]) + pb_ref[...]


_tail = pl.pallas_call(
    _tail_body,
    grid=(GN,),
    in_specs=[
        pl.BlockSpec((BN, DD), lambda i: (i, 0)),
        pl.BlockSpec((BN, DD), lambda i: (i, 0)),
        pl.BlockSpec((GN, 1, DD), lambda i: (0, 0, 0)),
        pl.BlockSpec((GN, 1, DD), lambda i: (0, 0, 0)),
        pl.BlockSpec((1, DD), lambda i: (0, 0)),
        pl.BlockSpec((DD, DD), lambda i: (0, 0)),
        pl.BlockSpec((1, DD), lambda i: (0, 0)),
        pl.BlockSpec((DD, DD), lambda i: (0, 0)),
        pl.BlockSpec((1, DD), lambda i: (0, 0)),
        pl.BlockSpec((DD, DD), lambda i: (0, 0)),
        pl.BlockSpec((1, DD), lambda i: (0, 0)),
        pl.BlockSpec((DD, DD), lambda i: (0, 0)),
        pl.BlockSpec((1, DD), lambda i: (0, 0)),
        pl.BlockSpec((1, DD), lambda i: (0, 0)),
        pl.BlockSpec((1, DD), lambda i: (0, 0)),
        pl.BlockSpec((1, DD), lambda i: (0, 0)),
        pl.BlockSpec((1, DD), lambda i: (0, 0)),
        pl.BlockSpec((1, DD), lambda i: (0, 0)),
        pl.BlockSpec((1, DD), lambda i: (0, 0)),
        pl.BlockSpec((FFD, DD), lambda i: (0, 0)),
        pl.BlockSpec((1, FFD), lambda i: (0, 0)),
        pl.BlockSpec((DD, FFD), lambda i: (0, 0)),
        pl.BlockSpec((1, DD), lambda i: (0, 0)),
        pl.BlockSpec((OUTD, DD), lambda i: (0, 0)),
        pl.BlockSpec((1, OUTD), lambda i: (0, 0)),
    ],
    out_specs=pl.BlockSpec((BN, OUTD), lambda i: (i, 0)),
    out_shape=jax.ShapeDtypeStruct((NN, OUTD), jnp.float32),
)


def kernel(feat, params, edge_index):
    p = params
    row = lambda v: v.reshape(1, -1)
    src_r = edge_index[0].reshape(NW, GCH, CHK)
    dst_r = edge_index[1].reshape(NW, GCH, CHK)
    zeros_nd = jnp.zeros((NN, DD), jnp.float32)

    h0 = _dense_in(feat, p["fc_in_w"], row(p["fc_in_b"]), p["w_cites"])

    parts0, deg_flat = _agg_call(h0, zeros_nd, src_r, dst_r, True)
    deg_r = deg_flat.reshape(NW, NN).T.reshape(GN, BN, NW)
    h1, ts1 = _layer(parts0, deg_r, p["gc_w"][0], row(p["gc_b"][0]),
                     jnp.full((1, DD), p["prelu_a"][0], jnp.float32),
                     p["attn_fc_w"], row(p["attn_fc_b"]))

    (parts1,) = _agg_call(h1, zeros_nd, src_r, dst_r, False)
    h2, ts2 = _layer(parts1, deg_r, p["gc_w"][1], row(p["gc_b"][1]),
                     jnp.full((1, DD), p["prelu_a"][1], jnp.float32),
                     p["attn_fc_w"], row(p["attn_fc_b"]))

    out = _tail(h1, h2, ts1, ts2, p["attn_vec"],
                p["sa_in_w"][2 * DD:], row(p["sa_in_b"][2 * DD:]),
                p["sa_out_w"], row(p["sa_out_b"]),
                p["ca_in_w"][2 * DD:], row(p["ca_in_b"][2 * DD:]),
                p["ca_out_w"], row(p["ca_out_b"]),
                row(p["ln1_w"]), row(p["ln1_b"]),
                row(p["ln2_w"]), row(p["ln2_b"]),
                row(p["ln3_w"]), row(p["ln3_b"]),
                p["ff1_w"].astype(jnp.bfloat16), row(p["ff1_b"]),
                p["ff2_w"].astype(jnp.bfloat16), row(p["ff2_b"]),
                p["pred_w"], row(p["pred_b"]))
    return out


# speedup vs baseline: 1.0452x; 1.0331x over previous
import functools

import jax
import jax.numpy as jnp
from jax import lax
from jax.experimental import pallas as pl
from jax.experimental.pallas import tpu as pltpu
from jax.experimental.pallas import tpu_sc as plsc

NN = 10000
EE = 320000
DD = 128
OUTD = 64
FFD = 2048

NC = 2
NS = 16
NW = NC * NS
EW = EE // NW
CHK = 80
GCH = EW // CHK
NBUF = 4
NBUF_DEG = 3
RPS = 624
TAIL = NN - NS * RPS

BN = 1000
GN = NN // BN
FCH = 512


def _agg_body(want_deg, h_hbm, src_hbm, dst_hbm, *rest):
    if want_deg:
        (out_hbm, deg_hbm, idx_s, idx_d, rows, accum, degp, gsem, isem) = rest
        nbuf = NBUF_DEG
    else:
        (out_hbm, idx_s, idx_d, rows, accum, gsem, isem) = rest
        nbuf = NBUF
    idxr = nbuf + 2
    c = lax.axis_index("c")
    s = lax.axis_index("s")
    wid = c * NS + s
    @pl.when(c == 0)
    def _():
        pltpu.sync_copy(h_hbm.at[pl.ds(s * RPS, RPS)],
                        accum.at[pl.ds(s * RPS, RPS)])
        @pl.when(s == NS - 1)
        def _():
            pltpu.sync_copy(h_hbm.at[pl.ds(NS * RPS, TAIL)],
                            accum.at[pl.ds(NS * RPS, TAIL)])
    @pl.when(c != 0)
    def _():
        z16 = jnp.zeros((16,), jnp.float32)

        def zrow(t, _):
            rows[0, t // (DD // 16), pl.ds((t % (DD // 16)) * 16, 16)] = z16
            return 0

        lax.fori_loop(0, CHK * (DD // 16), zrow, 0)
        for r in range(RPS // CHK):
            pltpu.sync_copy(rows.at[0],
                            accum.at[pl.ds(s * RPS + r * CHK, CHK)])
        rem = RPS % CHK
        if rem:
            pltpu.sync_copy(
                rows.at[0].at[pl.ds(0, rem)],
                accum.at[pl.ds(s * RPS + (RPS // CHK) * CHK, rem)])
        @pl.when(s == NS - 1)
        def _():
            pltpu.sync_copy(rows.at[0].at[pl.ds(0, TAIL)],
                            accum.at[pl.ds(NS * RPS, TAIL)])
    plsc.subcore_barrier()

    if want_deg:
        zeros16 = jnp.zeros((16,), jnp.float32)
        ones16 = jnp.ones((16,), jnp.float32)

        def zbody(i, _):
            degp[pl.ds(i * 16, 16)] = zeros16
            return 0

        lax.fori_loop(0, NN // 16, zbody, 0)

    def _idx_load(q):
        slot = lax.rem(q, idxr)
        pltpu.async_copy(src_hbm.at[wid, q], idx_s.at[slot], isem)
        pltpu.async_copy(dst_hbm.at[wid, q], idx_d.at[slot], isem)

    def _idx_wait(q):
        slot = lax.rem(q, idxr)
        pltpu.make_async_copy(src_hbm.at[wid, q], idx_s.at[slot], isem).wait()
        pltpu.make_async_copy(dst_hbm.at[wid, q], idx_d.at[slot], isem).wait()

    def _gather(q):
        pltpu.async_copy(h_hbm.at[idx_s.at[lax.rem(q, idxr)]],
                         rows.at[lax.rem(q, nbuf)], gsem)

    for r in range(idxr - 1):
        _idx_load(r)
    for j in range(nbuf - 1):
        _idx_wait(j)
        _gather(j)

    def body(g, _):
        q = g + nbuf - 1
        @pl.when(q < GCH)
        def _():
            @pl.when(g + idxr - 1 < GCH)
            def _():
                _idx_load(g + idxr - 1)
            _idx_wait(q)
            _gather(q)
        pltpu.make_async_copy(h_hbm.at[idx_s.at[lax.rem(g, idxr)]],
                              rows.at[lax.rem(g, nbuf)], gsem).wait()
        dslot = lax.rem(g, idxr)
        if want_deg:
            for k in range(CHK // 16):
                v = idx_d[dslot, pl.ds(k * 16, 16)]
                plsc.addupdate_scatter(degp, [v], ones16)
        pltpu.sync_copy(rows.at[lax.rem(g, nbuf)], accum.at[idx_d.at[dslot]],
                        add=True)
        return 0

    lax.fori_loop(0, GCH, body, 0)

    if want_deg:
        pltpu.sync_copy(degp, deg_hbm.at[pl.ds(wid * NN, NN)])

    plsc.subcore_barrier()
    pltpu.sync_copy(accum.at[pl.ds(s * RPS, RPS)],
                    out_hbm.at[c, pl.ds(s * RPS, RPS)])
    @pl.when(s == NS - 1)
    def _():
        pltpu.sync_copy(accum.at[pl.ds(NS * RPS, TAIL)],
                        out_hbm.at[c, pl.ds(NS * RPS, TAIL)])


@functools.lru_cache(maxsize=2)
def _build_agg_sc(want_deg):
    mesh = plsc.VectorSubcoreMesh(
        core_axis_name="c", subcore_axis_name="s",
        num_cores=NC, num_subcores=NS)
    nbuf = NBUF_DEG if want_deg else NBUF
    idxr = nbuf + 2
    out_type = [jax.ShapeDtypeStruct((NC, NN, DD), jnp.float32)]
    scratch = [
        pltpu.VMEM((idxr, CHK), jnp.int32),
        pltpu.VMEM((idxr, CHK), jnp.int32),
        pltpu.VMEM((nbuf, CHK, DD), jnp.float32),
        pltpu.VMEM_SHARED((NN, DD), jnp.float32),
    ]
    if want_deg:
        out_type.append(jax.ShapeDtypeStruct((NW * NN,), jnp.float32))
        scratch.append(pltpu.VMEM((NN,), jnp.float32))
    scratch.append(pltpu.SemaphoreType.DMA)
    scratch.append(pltpu.SemaphoreType.DMA)
    return pl.kernel(
        functools.partial(_agg_body, want_deg),
        out_type=out_type,
        mesh=mesh,
        scratch_types=scratch,
        compiler_params=pltpu.CompilerParams(
            use_tc_tiling_on_sc=False, needs_layout_passes=False),
    )


def _agg_call(h, src_r, dst_r, want_deg):
    return _build_agg_sc(want_deg)(h, src_r, dst_r)


def _dotT(x, w):
    return lax.dot_general(x, w, (((1,), (1,)), ((), ())),
                           preferred_element_type=jnp.float32)


def _dot(x, w):
    return lax.dot_general(x, w, (((1,), (0,)), ((), ())),
                           preferred_element_type=jnp.float32)


def _dense_in_body(feat_ref, w1_ref, b1_ref, w2_ref, out_ref):
    t = _dotT(feat_ref[...], w1_ref[...]) + b1_ref[...]
    out_ref[...] = _dot(t, w2_ref[...])


_dense_in = pl.pallas_call(
    _dense_in_body,
    grid=(GN,),
    in_specs=[
        pl.BlockSpec((BN, DD), lambda i: (i, 0)),
        pl.BlockSpec((DD, DD), lambda i: (0, 0)),
        pl.BlockSpec((1, DD), lambda i: (0, 0)),
        pl.BlockSpec((DD, DD), lambda i: (0, 0)),
    ],
    out_specs=pl.BlockSpec((BN, DD), lambda i: (i, 0)),
    out_shape=jax.ShapeDtypeStruct((NN, DD), jnp.float32),
)


def _layer_body(parts_ref, deg_ref, gw_ref, gb_ref, a_ref, aw_ref, ab_ref,
                h_ref, tsum_ref):
    p = parts_ref[0] + parts_ref[1]
    deg = jnp.sum(deg_ref[0], axis=1, keepdims=True) + 1.0
    hc = _dot(p * (1.0 / deg), gw_ref[...]) + gb_ref[...]
    hc = jnp.where(hc >= 0, hc, a_ref[...] * hc)
    t = jnp.tanh(_dotT(hc, aw_ref[...]) + ab_ref[...])
    tsum_ref[...] = jnp.sum(t, axis=0, keepdims=True).reshape(1, 1, DD)
    h_ref[...] = hc


_layer = pl.pallas_call(
    _layer_body,
    grid=(GN,),
    in_specs=[
        pl.BlockSpec((NC, BN, DD), lambda i: (0, i, 0)),
        pl.BlockSpec((1, BN, NW), lambda i: (i, 0, 0)),
        pl.BlockSpec((DD, DD), lambda i: (0, 0)),
        pl.BlockSpec((1, DD), lambda i: (0, 0)),
        pl.BlockSpec((1, DD), lambda i: (0, 0)),
        pl.BlockSpec((DD, DD), lambda i: (0, 0)),
        pl.BlockSpec((1, DD), lambda i: (0, 0)),
    ],
    out_specs=[
        pl.BlockSpec((BN, DD), lambda i: (i, 0)),
        pl.BlockSpec((1, 1, DD), lambda i: (i, 0, 0)),
    ],
    out_shape=[
        jax.ShapeDtypeStruct((NN, DD), jnp.float32),
        jax.ShapeDtypeStruct((GN, 1, DD), jnp.float32),
    ],
)


def _ln(x, w, b):
    mu = jnp.mean(x, axis=-1, keepdims=True)
    var = jnp.mean((x - mu) ** 2, axis=-1, keepdims=True)
    return (x - mu) / jnp.sqrt(var + 1e-5) * w + b


def _tail_body(h1_ref, h2_ref, ts1_ref, ts2_ref, av_ref,
               wv_sa_ref, bv_sa_ref, ow_sa_ref, ob_sa_ref,
               wv_ca_ref, bv_ca_ref, ow_ca_ref, ob_ca_ref,
               ln1w_ref, ln1b_ref, ln2w_ref, ln2b_ref, ln3w_ref, ln3b_ref,
               ff1_ref, ff1b_ref, ff2_ref, ff2b_ref,
               pw_ref, pb_ref, out_ref):
    av = av_ref[...]
    s1 = jnp.sum(jnp.sum(ts1_ref[...], axis=0) * av) / NN
    s2 = jnp.sum(jnp.sum(ts2_ref[...], axis=0) * av) / NN
    mx = jnp.maximum(s1, s2)
    e1 = jnp.exp(s1 - mx)
    e2 = jnp.exp(s2 - mx)
    b0 = e1 / (e1 + e2)
    b1 = e2 / (e1 + e2)
    z = b0 * h1_ref[...] + b1 * h2_ref[...]
    sa = _dotT(_dotT(z, wv_sa_ref[...]) + bv_sa_ref[...],
               ow_sa_ref[...]) + ob_sa_ref[...]
    x1 = _ln(z + sa, ln1w_ref[...], ln1b_ref[...])
    ca = _dotT(_dotT(z, wv_ca_ref[...]) + bv_ca_ref[...],
               ow_ca_ref[...]) + ob_ca_ref[...]
    x2 = _ln(x1 + ca, ln2w_ref[...], ln2b_ref[...])
    ff = jnp.zeros((BN, DD), jnp.float32) + ff2b_ref[...]
    x2b = x2.astype(jnp.bfloat16)
    for k in range(FFD // FCH):
        w1c = ff1_ref[k * FCH:(k + 1) * FCH, :]
        b1c = ff1b_ref[:, k * FCH:(k + 1) * FCH]
        hck = jnp.maximum(_dotT(x2b, w1c) + b1c, 0.0)
        ff = ff + _dotT(hck.astype(jnp.bfloat16),
                        ff2_ref[:, k * FCH:(k + 1) * FCH])
    x3 = _ln(x2 + ff, ln3w_ref[...], ln3b_ref[...])
    out_ref[...] = _dotT(x3, pw_ref[...]) + pb_ref[...]


_tail = pl.pallas_call(
    _tail_body,
    grid=(GN,),
    in_specs=[
        pl.BlockSpec((BN, DD), lambda i: (i, 0)),
        pl.BlockSpec((BN, DD), lambda i: (i, 0)),
        pl.BlockSpec((GN, 1, DD), lambda i: (0, 0, 0)),
        pl.BlockSpec((GN, 1, DD), lambda i: (0, 0, 0)),
        pl.BlockSpec((1, DD), lambda i: (0, 0)),
        pl.BlockSpec((DD, DD), lambda i: (0, 0)),
        pl.BlockSpec((1, DD), lambda i: (0, 0)),
        pl.BlockSpec((DD, DD), lambda i: (0, 0)),
        pl.BlockSpec((1, DD), lambda i: (0, 0)),
        pl.BlockSpec((DD, DD), lambda i: (0, 0)),
        pl.BlockSpec((1, DD), lambda i: (0, 0)),
        pl.BlockSpec((DD, DD), lambda i: (0, 0)),
        pl.BlockSpec((1, DD), lambda i: (0, 0)),
        pl.BlockSpec((1, DD), lambda i: (0, 0)),
        pl.BlockSpec((1, DD), lambda i: (0, 0)),
        pl.BlockSpec((1, DD), lambda i: (0, 0)),
        pl.BlockSpec((1, DD), lambda i: (0, 0)),
        pl.BlockSpec((1, DD), lambda i: (0, 0)),
        pl.BlockSpec((1, DD), lambda i: (0, 0)),
        pl.BlockSpec((FFD, DD), lambda i: (0, 0)),
        pl.BlockSpec((1, FFD), lambda i: (0, 0)),
        pl.BlockSpec((DD, FFD), lambda i: (0, 0)),
        pl.BlockSpec((1, DD), lambda i: (0, 0)),
        pl.BlockSpec((OUTD, DD), lambda i: (0, 0)),
        pl.BlockSpec((1, OUTD), lambda i: (0, 0)),
    ],
    out_specs=pl.BlockSpec((BN, OUTD), lambda i: (i, 0)),
    out_shape=jax.ShapeDtypeStruct((NN, OUTD), jnp.float32),
)


def kernel(feat, params, edge_index):
    p = params
    row = lambda v: v.reshape(1, -1)
    src_r = edge_index[0].reshape(NW, GCH, CHK)
    dst_r = edge_index[1].reshape(NW, GCH, CHK)

    h0 = _dense_in(feat, p["fc_in_w"], row(p["fc_in_b"]), p["w_cites"])

    parts0, deg_flat = _agg_call(h0, src_r, dst_r, True)
    deg_r = deg_flat.reshape(NW, NN).T.reshape(GN, BN, NW)
    h1, ts1 = _layer(parts0, deg_r, p["gc_w"][0], row(p["gc_b"][0]),
                     jnp.full((1, DD), p["prelu_a"][0], jnp.float32),
                     p["attn_fc_w"], row(p["attn_fc_b"]))

    (parts1,) = _agg_call(h1, src_r, dst_r, False)
    h2, ts2 = _layer(parts1, deg_r, p["gc_w"][1], row(p["gc_b"][1]),
                     jnp.full((1, DD), p["prelu_a"][1], jnp.float32),
                     p["attn_fc_w"], row(p["attn_fc_b"]))

    out = _tail(h1, h2, ts1, ts2, p["attn_vec"],
                p["sa_in_w"][2 * DD:], row(p["sa_in_b"][2 * DD:]),
                p["sa_out_w"], row(p["sa_out_b"]),
                p["ca_in_w"][2 * DD:], row(p["ca_in_b"][2 * DD:]),
                p["ca_out_w"], row(p["ca_out_b"]),
                row(p["ln1_w"]), row(p["ln1_b"]),
                row(p["ln2_w"]), row(p["ln2_b"]),
                row(p["ln3_w"]), row(p["ln3_b"]),
                p["ff1_w"].astype(jnp.bfloat16), row(p["ff1_b"]),
                p["ff2_w"].astype(jnp.bfloat16), row(p["ff2_b"]),
                p["pred_w"], row(p["pred_b"]))
    return out
